# Initial kernel scaffold; baseline (speedup 1.0000x reference)
import jax
import jax.numpy as jnp
from jax.experimental import pallas as pl


def kernel(feature_ids, time_ids, code_type_ids, feature_table, time_table, code_type_table, gamma, beta):
    raise NotImplementedError("write your pallas kernel here")


# speedup vs baseline: 2.4549x; 2.4549x over previous
import functools

import jax
import jax.numpy as jnp
from jax import lax
from jax.experimental import pallas as pl
from jax.experimental.pallas import tpu as pltpu
from jax.experimental.pallas import tpu_sc as plsc

H = 128
EPS = 1e-12
NC = 2
NS = 16
NW = NC * NS
T = 256
G = 128


def _rsqrt(x):
  i = lax.bitcast_convert_type(x, jnp.int32)
  i = jnp.int32(0x5F3759DF) - lax.shift_right_arithmetic(i, jnp.int32(1))
  y = lax.bitcast_convert_type(i, jnp.float32)
  for _ in range(3):
    y = y * (1.5 - 0.5 * x * y * y)
  return y


def _body(fid_hbm, tid_hbm, cid_hbm, ftab_hbm, ttab_hbm, ctab_hbm,
          gamma_hbm, beta_hbm, out_hbm,
          fid_v, tid_v, cid_v, rf, rt, rc, gb_v, semf, semt, semc):
  n_tokens = fid_hbm.shape[0]
  n_per_w = n_tokens // NW
  n_chunks = n_per_w // T
  rows_per_chunk = T // G

  wid = lax.axis_index("s") * NC + lax.axis_index("c")

  pltpu.sync_copy(gamma_hbm, gb_v.at[0])
  pltpu.sync_copy(beta_hbm, gb_v.at[1])

  def chunk_body(k, _):
    tok0 = wid * n_per_w + k * T

    pltpu.sync_copy(fid_hbm.at[pl.ds(tok0, T)], fid_v)
    pltpu.sync_copy(tid_hbm.at[pl.ds(tok0, T)], tid_v)
    pltpu.sync_copy(cid_hbm.at[pl.ds(tok0, T)], cid_v)

    copies = []
    for j in range(rows_per_chunk):
      copies.append(pltpu.async_copy(
          ftab_hbm.at[fid_v.at[pl.ds(j * G, G)]], rf.at[pl.ds(j * G, G)],
          semf))
      copies.append(pltpu.async_copy(
          ttab_hbm.at[tid_v.at[pl.ds(j * G, G)]], rt.at[pl.ds(j * G, G)],
          semt))
      copies.append(pltpu.async_copy(
          ctab_hbm.at[cid_v.at[pl.ds(j * G, G)]], rc.at[pl.ds(j * G, G)],
          semc))
    for c in copies:
      c.wait()

    def token_body(i, _):
      ii = i
      accs = []
      s = jnp.zeros((16,), jnp.float32)
      ss = jnp.zeros((16,), jnp.float32)
      for j in range(H // 16):
        a = (rf[ii, pl.ds(16 * j, 16)] + rt[ii, pl.ds(16 * j, 16)]
             + rc[ii, pl.ds(16 * j, 16)])
        accs.append(a)
        s = s + a
        ss = ss + a * a
      tot = jnp.sum(s)
      tot2 = jnp.sum(ss)
      mean = tot * (1.0 / H)
      var = tot2 * (1.0 / H) - mean * mean
      rstd = _rsqrt(var + EPS)
      for j in range(H // 16):
        g = gb_v[0, pl.ds(16 * j, 16)]
        b = gb_v[1, pl.ds(16 * j, 16)]
        rf[ii, pl.ds(16 * j, 16)] = (accs[j] - mean) * rstd * g + b
      return 0

    lax.fori_loop(0, T, token_body, 0)

    pltpu.sync_copy(rf, out_hbm.at[pl.ds(tok0, T)])
    return 0

  lax.fori_loop(0, n_chunks, chunk_body, 0)


def kernel(feature_ids, time_ids, code_type_ids, feature_table, time_table,
           code_type_table, gamma, beta):
  B, L = feature_ids.shape
  N = B * L
  fid = feature_ids.reshape(N).astype(jnp.int32)
  tid = time_ids.reshape(N).astype(jnp.int32)
  cid = code_type_ids.reshape(N).astype(jnp.int32)

  mesh = plsc.VectorSubcoreMesh(core_axis_name="c", subcore_axis_name="s")
  run = pl.kernel(
      _body,
      out_type=jax.ShapeDtypeStruct((N, H), jnp.float32),
      mesh=mesh,
      compiler_params=pltpu.CompilerParams(needs_layout_passes=False),
      scratch_types=[
          pltpu.VMEM((T,), jnp.int32),
          pltpu.VMEM((T,), jnp.int32),
          pltpu.VMEM((T,), jnp.int32),
          pltpu.VMEM((T, H), jnp.float32),
          pltpu.VMEM((T, H), jnp.float32),
          pltpu.VMEM((T, H), jnp.float32),
          pltpu.VMEM((2, H), jnp.float32),
          pltpu.SemaphoreType.DMA,
          pltpu.SemaphoreType.DMA,
          pltpu.SemaphoreType.DMA,
      ],
  )
  out = run(fid, tid, cid, feature_table, time_table, code_type_table,
            gamma, beta)
  return out.reshape(B, L, H)


# speedup vs baseline: 2.6861x; 1.0942x over previous
import jax
import jax.numpy as jnp
from jax import lax
from jax.experimental import pallas as pl
from jax.experimental.pallas import tpu as pltpu
from jax.experimental.pallas import tpu_sc as plsc

H = 128
EPS = 1e-12
NC = 2
NS = 16
NW = NC * NS
T = 128
NBUF = 2


def _rsqrt(x):
  i = lax.bitcast_convert_type(x, jnp.int32)
  i = jnp.int32(0x5F3759DF) - lax.shift_right_arithmetic(i, jnp.int32(1))
  y = lax.bitcast_convert_type(i, jnp.float32)
  for _ in range(2):
    y = y * (1.5 - 0.5 * x * y * y)
  return y


def _tree_sum(vs):
  while len(vs) > 1:
    vs = [a + b for a, b in zip(vs[::2], vs[1::2])]
  return vs[0]


def _body(fid_hbm, tid_hbm, cid_hbm, ftab_hbm, ttab_hbm, ctab_hbm,
          gamma_hbm, beta_hbm, out_hbm,
          idx_v, rows_v, gb_v, sems, semo):
  n_tokens = fid_hbm.shape[0]
  n_per_w = n_tokens // NW
  n_chunks = n_per_w // T

  wid = lax.axis_index("s") * NC + lax.axis_index("c")
  base = wid * n_per_w

  pltpu.sync_copy(gamma_hbm, gb_v.at[0])
  pltpu.sync_copy(beta_hbm, gb_v.at[1])

  tabs = (ftab_hbm, ttab_hbm, ctab_hbm)
  ids = (fid_hbm, tid_hbm, cid_hbm)

  def fire(b, k):
    tok0 = base + k * T
    for t in range(3):
      pltpu.sync_copy(ids[t].at[pl.ds(tok0, T)], idx_v.at[b].at[t])
    for t in range(3):
      pltpu.async_copy(tabs[t].at[idx_v.at[b].at[t]], rows_v.at[b].at[t],
                       sems.at[b])

  def wait_gathers(b):
    for t in range(3):
      pltpu.make_async_copy(tabs[t].at[idx_v.at[b].at[t]],
                            rows_v.at[b].at[t], sems.at[b]).wait()

  def compute(b, k):
    rf = rows_v.at[b].at[0]
    rt = rows_v.at[b].at[1]
    rc = rows_v.at[b].at[2]

    @plsc.parallel_loop(0, T, unroll=4)
    def token_body(i):
      accs = []
      for j in range(H // 16):
        accs.append(rf[i, pl.ds(16 * j, 16)] + rt[i, pl.ds(16 * j, 16)]
                    + rc[i, pl.ds(16 * j, 16)])
      s = _tree_sum(accs)
      ss = _tree_sum([a * a for a in accs])
      tot = jnp.sum(s)
      tot2 = jnp.sum(ss)
      mean = tot * (1.0 / H)
      var = tot2 * (1.0 / H) - mean * mean
      rstd = _rsqrt(var + EPS)
      mrstd = mean * rstd
      for j in range(H // 16):
        g = gb_v[0, pl.ds(16 * j, 16)]
        bta = gb_v[1, pl.ds(16 * j, 16)]
        rf[i, pl.ds(16 * j, 16)] = (accs[j] * rstd - mrstd) * g + bta

    pltpu.sync_copy(rf, out_hbm.at[pl.ds(base + k * T, T)])

  fire(0, 0)

  def outer(k2, _):
    for b in range(NBUF):
      k = k2 * NBUF + b
      wait_gathers(b)
      nk = k + 1

      @pl.when(nk < n_chunks)
      def _():
        fire((b + 1) % NBUF, nk)

      compute(b, k)
    return 0

  lax.fori_loop(0, n_chunks // NBUF, outer, 0)


def kernel(feature_ids, time_ids, code_type_ids, feature_table, time_table,
           code_type_table, gamma, beta):
  B, L = feature_ids.shape
  N = B * L
  fid = feature_ids.reshape(N).astype(jnp.int32)
  tid = time_ids.reshape(N).astype(jnp.int32)
  cid = code_type_ids.reshape(N).astype(jnp.int32)

  mesh = plsc.VectorSubcoreMesh(core_axis_name="c", subcore_axis_name="s")
  run = pl.kernel(
      _body,
      out_type=jax.ShapeDtypeStruct((N, H), jnp.float32),
      mesh=mesh,
      compiler_params=pltpu.CompilerParams(needs_layout_passes=False),
      scratch_types=[
          pltpu.VMEM((NBUF, 3, T), jnp.int32),
          pltpu.VMEM((NBUF, 3, T, H), jnp.float32),
          pltpu.VMEM((2, H), jnp.float32),
          pltpu.SemaphoreType.DMA((NBUF,)),
          pltpu.SemaphoreType.DMA,
      ],
  )
  out = run(fid, tid, cid, feature_table, time_table, code_type_table,
            gamma, beta)
  return out.reshape(B, L, H)
